# baseline (speedup 1.0000x reference)
import functools

import jax
import jax.numpy as jnp
from jax.experimental import pallas as pl

N_TOKENS = 8192
D_MODEL = 768
N_EXPERTS = 8
BLOCK_T = 1024


def _moe_body(x_ref, gw_ref, ew_ref, eb_ref, o_ref):
    x = x_ref[...]
    logits = jnp.dot(x, gw_ref[...], preferred_element_type=jnp.float32)
    w = jax.nn.softmax(logits, axis=-1)
    acc0 = jnp.dot(w, eb_ref[...], preferred_element_type=jnp.float32)
    acc1 = jnp.zeros_like(acc0)
    xb = x.astype(jnp.bfloat16)
    for e in range(0, N_EXPERTS, 2):
        y0 = jnp.dot(xb, ew_ref[e], preferred_element_type=jnp.float32)
        y1 = jnp.dot(xb, ew_ref[e + 1], preferred_element_type=jnp.float32)
        acc0 = acc0 + w[:, e : e + 1] * y0
        acc1 = acc1 + w[:, e + 1 : e + 2] * y1
    o_ref[...] = (acc0 + acc1).astype(o_ref.dtype)


@functools.partial(jax.jit, static_argnames=("interpret",))
def kernel(inputs, gate_w, expert_w, expert_b, interpret=False):
    n_tokens, d_model = inputs.shape
    n_experts = expert_w.shape[0]
    grid = (n_tokens // BLOCK_T,)
    return pl.pallas_call(
        _moe_body,
        grid=grid,
        in_specs=[
            pl.BlockSpec((BLOCK_T, d_model), lambda i: (i, 0)),
            pl.BlockSpec((d_model, n_experts), lambda i: (0, 0)),
            pl.BlockSpec((n_experts, d_model, d_model), lambda i: (0, 0, 0)),
            pl.BlockSpec((n_experts, d_model), lambda i: (0, 0)),
        ],
        out_specs=pl.BlockSpec((BLOCK_T, d_model), lambda i: (i, 0)),
        out_shape=jax.ShapeDtypeStruct((n_tokens, d_model), inputs.dtype),
        interpret=interpret,
    )(inputs, gate_w, expert_w, expert_b)
